# baseline (speedup 1.0000x reference)
import functools

import jax
import jax.numpy as jnp
from jax import lax
from jax.experimental import pallas as pl
from jax.experimental.pallas import tpu as pltpu
from jax.experimental.pallas import tpu_sc as plsc

R = 16384
C = 50
B = R * C
D = 64
NC = 2
NS = 16
NW = NC * NS
TB = 128
NT = R // TB
T_PER_W = NT // NW
XBLK = TB * C

_mesh = plsc.VectorSubcoreMesh(core_axis_name="c", subcore_axis_name="s")


@functools.partial(
    pl.kernel,
    mesh=_mesh,
    out_type=jax.ShapeDtypeStruct((C, D // 8, NT, 8, TB), jnp.float32),
    scratch_types=[
        pltpu.VMEM((XBLK,), jnp.int32),
        pltpu.VMEM((C, TB), jnp.int32),
        pltpu.VMEM((TB, D), jnp.float32),
        pltpu.VMEM((TB, D), jnp.float32),
        pltpu.VMEM((D, TB), jnp.float32),
        pltpu.VMEM((D, TB), jnp.float32),
        pltpu.SemaphoreType.DMA((2,)),
        pltpu.SemaphoreType.DMA((2,)),
    ],
    compiler_params=pltpu.CompilerParams(
        use_tc_tiling_on_sc=False, needs_layout_passes=False
    ),
)
def _gather(idx_hbm, table_hbm, out_hbm, xv, idxT, g0, g1, t0, t1, gsem, ssem):
    rows = [g0, g1]
    tbufs = [t0, t1]
    wid = lax.axis_index("s") * NC + lax.axis_index("c")
    lanes = lax.iota(jnp.int32, 16)

    def start_gather(p, j):
        pltpu.async_copy(table_hbm.at[idxT.at[j]], rows[p], gsem.at[p])

    def wait_gather(p, j):
        pltpu.make_async_copy(table_hbm.at[idxT.at[j]], rows[p], gsem.at[p]).wait()

    def start_store(p, j, b):
        for a in range(D // 8):
            pltpu.async_copy(
                tbufs[p].at[pl.ds(a * 8, 8)], out_hbm.at[j, a, b], ssem.at[p]
            )

    def wait_store(p, j, b):
        for a in range(D // 8):
            pltpu.make_async_copy(
                tbufs[p].at[pl.ds(a * 8, 8)], out_hbm.at[j, a, b], ssem.at[p]
            ).wait()

    def transpose_block(p):
        rbuf, tbuf = rows[p], tbufs[p]

        @pl.loop(0, D, step=8)
        def _(k0):
            for dk in range(8):
                k = k0 + dk
                kv = jnp.full((16,), k, jnp.int32)
                for icblk in range(TB // 16):
                    col = plsc.load_gather(rbuf, [lanes + icblk * 16, kv])
                    tbuf[k, pl.ds(icblk * 16, 16)] = col

    for t in range(T_PER_W):
        b = wid * T_PER_W + t
        pltpu.sync_copy(idx_hbm.at[pl.ds(b * XBLK, XBLK)], xv)

        def jbody(j, _):
            for icblk in range(TB // 16):
                pos = lanes * C + (icblk * 16 * C + j)
                v = plsc.load_gather(xv, [pos])
                idxT[j, pl.ds(icblk * 16, 16)] = v
            return 0

        lax.fori_loop(0, C, jbody, 0)

        start_gather(0, 0)

        @pl.loop(0, C, step=2)
        def _(j):
            for p in range(2):
                jj = j + p
                wait_gather(p, jj)

                @pl.when(jj + 1 < C)
                def _():
                    start_gather(1 - p, jj + 1)

                @pl.when(jj >= 2)
                def _():
                    wait_store(p, jj - 2, b)

                transpose_block(p)
                start_store(p, jj, b)

        wait_store(0, C - 2, b)
        wait_store(1, C - 1, b)


def kernel(x, weight):
    xf = x.reshape(-1).astype(jnp.int32)
    y5 = _gather(xf, weight)
    return y5.transpose(2, 4, 0, 1, 3).reshape(R, C, D)


# speedup vs baseline: 1.8470x; 1.8470x over previous
import functools

import jax
import jax.numpy as jnp
from jax import lax
from jax.experimental import pallas as pl
from jax.experimental.pallas import tpu as pltpu
from jax.experimental.pallas import tpu_sc as plsc

R = 16384
C = 50
B = R * C
D = 64
NC = 2
NS = 16
NW = NC * NS
TB = 128
NT = R // TB
T_PER_W = NT // NW
XBLK = TB * C

_mesh = plsc.VectorSubcoreMesh(core_axis_name="c", subcore_axis_name="s")


@functools.partial(
    pl.kernel,
    mesh=_mesh,
    out_type=jax.ShapeDtypeStruct((C, D // 8, NT, 8, TB), jnp.float32),
    scratch_types=[
        pltpu.VMEM((XBLK,), jnp.int32),
        pltpu.VMEM((C, TB), jnp.int32),
        pltpu.VMEM((TB, D), jnp.float32),
        pltpu.VMEM((TB, D), jnp.float32),
        pltpu.VMEM((D, TB + 1), jnp.float32),
        pltpu.VMEM((D, TB + 1), jnp.float32),
        pltpu.SemaphoreType.DMA((2,)),
        pltpu.SemaphoreType.DMA((2,)),
    ],
    compiler_params=pltpu.CompilerParams(
        use_tc_tiling_on_sc=False, needs_layout_passes=False
    ),
)
def _gather(idx_hbm, table_hbm, out_hbm, xv, idxT, g0, g1, t0, t1, gsem, ssem):
    rows = [g0, g1]
    tbufs = [t0, t1]
    wid = lax.axis_index("s") * NC + lax.axis_index("c")
    lanes = lax.iota(jnp.int32, 16)

    def start_gather(p, j):
        pltpu.async_copy(table_hbm.at[idxT.at[j]], rows[p], gsem.at[p])

    def wait_gather(p, j):
        pltpu.make_async_copy(table_hbm.at[idxT.at[j]], rows[p], gsem.at[p]).wait()

    def start_store(p, j, b):
        for a in range(D // 8):
            pltpu.async_copy(
                tbufs[p].at[pl.ds(a * 8, 8), pl.ds(0, TB)],
                out_hbm.at[j, a, b],
                ssem.at[p],
            )

    def wait_store(p, j, b):
        for a in range(D // 8):
            pltpu.make_async_copy(
                tbufs[p].at[pl.ds(a * 8, 8), pl.ds(0, TB)],
                out_hbm.at[j, a, b],
                ssem.at[p],
            ).wait()

    def transpose_block(p):
        rbuf, tbuf = rows[p], tbufs[p]

        @pl.loop(0, TB, step=4)
        def _(ic0):
            for dic in range(4):
                ic = ic0 + dic
                icv = jnp.full((16,), ic, jnp.int32)
                for kblk in range(D // 16):
                    v = rbuf[ic, pl.ds(kblk * 16, 16)]
                    plsc.store_scatter(tbuf, [lanes + kblk * 16, icv], v)

    for t in range(T_PER_W):
        b = wid * T_PER_W + t
        pltpu.sync_copy(idx_hbm.at[pl.ds(b * XBLK, XBLK)], xv)

        def jbody(j, _):
            for icblk in range(TB // 16):
                pos = lanes * C + (icblk * 16 * C + j)
                v = plsc.load_gather(xv, [pos])
                idxT[j, pl.ds(icblk * 16, 16)] = v
            return 0

        lax.fori_loop(0, C, jbody, 0)

        start_gather(0, 0)

        @pl.loop(0, C, step=2)
        def _(j):
            for p in range(2):
                jj = j + p
                wait_gather(p, jj)

                @pl.when(jj + 1 < C)
                def _():
                    start_gather(1 - p, jj + 1)

                @pl.when(jj >= 2)
                def _():
                    wait_store(p, jj - 2, b)

                transpose_block(p)
                start_store(p, jj, b)

        wait_store(0, C - 2, b)
        wait_store(1, C - 1, b)


def kernel(x, weight):
    xf = x.reshape(-1).astype(jnp.int32)
    y5 = _gather(xf, weight)
    return y5.transpose(2, 4, 0, 1, 3).reshape(R, C, D)


# speedup vs baseline: 1.8525x; 1.0030x over previous
import functools

import jax
import jax.numpy as jnp
from jax import lax
from jax.experimental import pallas as pl
from jax.experimental.pallas import tpu as pltpu
from jax.experimental.pallas import tpu_sc as plsc

R = 16384
C = 50
B = R * C
D = 64
NC = 2
NS = 16
NW = NC * NS
TB = 128
NT = R // TB
T_PER_W = NT // NW
XBLK = TB * C

_mesh = plsc.VectorSubcoreMesh(core_axis_name="c", subcore_axis_name="s")


@functools.partial(
    pl.kernel,
    mesh=_mesh,
    out_type=jax.ShapeDtypeStruct((C, D // 8, NT, 8, TB), jnp.float32),
    scratch_types=[
        pltpu.VMEM((XBLK,), jnp.int32),
        pltpu.VMEM((C, TB), jnp.int32),
        pltpu.VMEM((TB, D), jnp.float32),
        pltpu.VMEM((TB, D), jnp.float32),
        pltpu.VMEM((D, TB + 1), jnp.float32),
        pltpu.VMEM((D, TB + 1), jnp.float32),
        pltpu.SemaphoreType.DMA((2,)),
        pltpu.SemaphoreType.DMA((2,)),
    ],
    compiler_params=pltpu.CompilerParams(
        use_tc_tiling_on_sc=False, needs_layout_passes=False
    ),
)
def _gather(idx_hbm, table_hbm, out_hbm, xv, idxT, g0, g1, t0, t1, gsem, ssem):
    rows = [g0, g1]
    tbufs = [t0, t1]
    wid = lax.axis_index("s") * NC + lax.axis_index("c")
    lanes = lax.iota(jnp.int32, 16)

    def start_gather(p, j):
        pltpu.async_copy(table_hbm.at[idxT.at[j]], rows[p], gsem.at[p])

    def wait_gather(p, j):
        pltpu.make_async_copy(table_hbm.at[idxT.at[j]], rows[p], gsem.at[p]).wait()

    def start_store(p, j, b):
        for a in range(D // 8):
            pltpu.async_copy(
                tbufs[p].at[pl.ds(a * 8, 8), pl.ds(0, TB)],
                out_hbm.at[j, a, b],
                ssem.at[p],
            )

    def wait_store(p, j, b):
        for a in range(D // 8):
            pltpu.make_async_copy(
                tbufs[p].at[pl.ds(a * 8, 8), pl.ds(0, TB)],
                out_hbm.at[j, a, b],
                ssem.at[p],
            ).wait()

    def transpose_block(p):
        rbuf, tbuf = rows[p], tbufs[p]

        @pl.loop(0, TB, step=8)
        def _(ic0):
            for dic in range(8):
                ic = ic0 + dic
                icv = jnp.full((16,), ic, jnp.int32)
                for kblk in range(D // 16):
                    v = rbuf[ic, pl.ds(kblk * 16, 16)]
                    plsc.store_scatter(tbuf, [lanes + kblk * 16, icv], v)

    for t in range(T_PER_W):
        b = wid * T_PER_W + t
        pltpu.sync_copy(idx_hbm.at[pl.ds(b * XBLK, XBLK)], xv)

        def jbody(j, _):
            for icblk in range(TB // 16):
                pos = lanes * C + (icblk * 16 * C + j)
                v = plsc.load_gather(xv, [pos])
                idxT[j, pl.ds(icblk * 16, 16)] = v
            return 0

        lax.fori_loop(0, C, jbody, 0)

        start_gather(0, 0)

        @pl.loop(0, C, step=2)
        def _(j):
            for p in range(2):
                jj = j + p
                wait_gather(p, jj)

                @pl.when(jj + 1 < C)
                def _():
                    start_gather(1 - p, jj + 1)

                @pl.when(jj >= 2)
                def _():
                    wait_store(p, jj - 2, b)

                transpose_block(p)
                start_store(p, jj, b)

        wait_store(0, C - 2, b)
        wait_store(1, C - 1, b)


def kernel(x, weight):
    xf = x.reshape(-1).astype(jnp.int32)
    y5 = _gather(xf, weight)
    return y5.transpose(2, 4, 0, 1, 3).reshape(R, C, D)


# speedup vs baseline: 2.1794x; 1.1765x over previous
import functools

import jax
import jax.numpy as jnp
from jax import lax
from jax.experimental import pallas as pl
from jax.experimental.pallas import tpu as pltpu
from jax.experimental.pallas import tpu_sc as plsc

R = 16384
C = 50
B = R * C
D = 64
NC = 2
NS = 16
NW = NC * NS
TB = 128
NT = R // TB
T_PER_W = NT // NW
XBLK = TB * C

_mesh = plsc.VectorSubcoreMesh(core_axis_name="c", subcore_axis_name="s")


@functools.partial(
    pl.kernel,
    mesh=_mesh,
    out_type=jax.ShapeDtypeStruct((C, D // 8, NT, 8, TB), jnp.float32),
    scratch_types=[
        pltpu.VMEM((XBLK,), jnp.int32),
        pltpu.VMEM((C, TB), jnp.int32),
        pltpu.VMEM((TB, D), jnp.float32),
        pltpu.VMEM((TB, D), jnp.float32),
        pltpu.VMEM((D, TB + 1), jnp.float32),
        pltpu.VMEM((D, TB + 1), jnp.float32),
        pltpu.SemaphoreType.DMA((2,)),
        pltpu.SemaphoreType.DMA((2,)),
    ],
    compiler_params=pltpu.CompilerParams(
        use_tc_tiling_on_sc=False, needs_layout_passes=False
    ),
)
def _gather(idx_hbm, table_hbm, out_hbm, xv, idxT, g0, g1, t0, t1, gsem, ssem):
    rows = [g0, g1]
    tbufs = [t0, t1]
    wid = lax.axis_index("s") * NC + lax.axis_index("c")
    lanes = lax.iota(jnp.int32, 16)

    def start_gather(p, j):
        pltpu.async_copy(table_hbm.at[idxT.at[j]], rows[p], gsem.at[p])

    def wait_gather(p, j):
        pltpu.make_async_copy(table_hbm.at[idxT.at[j]], rows[p], gsem.at[p]).wait()

    def start_store(p, j, b):
        for a in range(D // 8):
            pltpu.async_copy(
                tbufs[p].at[pl.ds(a * 8, 8), pl.ds(0, TB)],
                out_hbm.at[j, a, b],
                ssem.at[p],
            )

    def wait_store(p, j, b):
        for a in range(D // 8):
            pltpu.make_async_copy(
                tbufs[p].at[pl.ds(a * 8, 8), pl.ds(0, TB)],
                out_hbm.at[j, a, b],
                ssem.at[p],
            ).wait()

    def transpose_block(p):
        rbuf, tbuf = rows[p], tbufs[p]

        @plsc.parallel_loop(0, TB, step=8)
        def _(ic0):
            for dic in range(8):
                ic = ic0 + dic
                icv = jnp.full((16,), ic, jnp.int32)
                for kblk in range(D // 16):
                    v = rbuf[ic, pl.ds(kblk * 16, 16)]
                    plsc.store_scatter(tbuf, [lanes + kblk * 16, icv], v)

    for t in range(T_PER_W):
        b = wid * T_PER_W + t
        pltpu.sync_copy(idx_hbm.at[pl.ds(b * XBLK, XBLK)], xv)

        def jbody(j, _):
            for icblk in range(TB // 16):
                pos = lanes * C + (icblk * 16 * C + j)
                v = plsc.load_gather(xv, [pos])
                idxT[j, pl.ds(icblk * 16, 16)] = v
            return 0

        lax.fori_loop(0, C, jbody, 0)

        start_gather(0, 0)

        @pl.loop(0, C, step=2)
        def _(j):
            for p in range(2):
                jj = j + p
                wait_gather(p, jj)

                @pl.when(jj + 1 < C)
                def _():
                    start_gather(1 - p, jj + 1)

                @pl.when(jj >= 2)
                def _():
                    wait_store(p, jj - 2, b)

                transpose_block(p)
                start_store(p, jj, b)

        wait_store(0, C - 2, b)
        wait_store(1, C - 1, b)


def kernel(x, weight):
    xf = x.reshape(-1).astype(jnp.int32)
    y5 = _gather(xf, weight)
    return y5.transpose(2, 4, 0, 1, 3).reshape(R, C, D)


# speedup vs baseline: 2.2007x; 1.0098x over previous
import functools

import jax
import jax.numpy as jnp
from jax import lax
from jax.experimental import pallas as pl
from jax.experimental.pallas import tpu as pltpu
from jax.experimental.pallas import tpu_sc as plsc

R = 16384
C = 50
B = R * C
D = 64
NC = 2
NS = 16
NW = NC * NS
TB = 128
NT = R // TB
T_PER_W = NT // NW
XBLK = TB * C

_mesh = plsc.VectorSubcoreMesh(core_axis_name="c", subcore_axis_name="s")


@functools.partial(
    pl.kernel,
    mesh=_mesh,
    out_type=jax.ShapeDtypeStruct((C, D // 8, NT, 8, TB), jnp.float32),
    scratch_types=[
        pltpu.VMEM((XBLK,), jnp.int32),
        pltpu.VMEM((C, TB), jnp.int32),
        pltpu.VMEM((TB, D), jnp.float32),
        pltpu.VMEM((TB, D), jnp.float32),
        pltpu.VMEM((D, TB + 1), jnp.float32),
        pltpu.VMEM((D, TB + 1), jnp.float32),
        pltpu.SemaphoreType.DMA((2,)),
        pltpu.SemaphoreType.DMA((2,)),
    ],
    compiler_params=pltpu.CompilerParams(
        use_tc_tiling_on_sc=False, needs_layout_passes=False
    ),
)
def _gather(idx_hbm, table_hbm, out_hbm, xv, idxT, g0, g1, t0, t1, gsem, ssem):
    rows = [g0, g1]
    tbufs = [t0, t1]
    wid = lax.axis_index("s") * NC + lax.axis_index("c")
    lanes = lax.iota(jnp.int32, 16)

    def start_gather(p, j):
        pltpu.async_copy(table_hbm.at[idxT.at[j]], rows[p], gsem.at[p])

    def wait_gather(p, j):
        pltpu.make_async_copy(table_hbm.at[idxT.at[j]], rows[p], gsem.at[p]).wait()

    def start_store(p, j, b):
        for a in range(D // 8):
            pltpu.async_copy(
                tbufs[p].at[pl.ds(a * 8, 8), pl.ds(0, TB)],
                out_hbm.at[j, a, b],
                ssem.at[p],
            )

    def wait_store(p, j, b):
        for a in range(D // 8):
            pltpu.make_async_copy(
                tbufs[p].at[pl.ds(a * 8, 8), pl.ds(0, TB)],
                out_hbm.at[j, a, b],
                ssem.at[p],
            ).wait()

    def transpose_block(p):
        rbuf, tbuf = rows[p], tbufs[p]

        @plsc.parallel_loop(0, TB, step=8)
        def _(ic0):
            for dic in range(8):
                ic = ic0 + dic
                icv = jnp.full((16,), ic, jnp.int32)
                for kblk in range(D // 16):
                    v = rbuf[ic, pl.ds(kblk * 16, 16)]
                    plsc.store_scatter(tbuf, [lanes + kblk * 16, icv], v)

    for t in range(T_PER_W):
        b = wid * T_PER_W + t
        pltpu.sync_copy(idx_hbm.at[pl.ds(b * XBLK, XBLK)], xv)

        @plsc.parallel_loop(0, C, step=2)
        def _(j0):
            for dj in range(2):
                j = j0 + dj
                for icblk in range(TB // 16):
                    pos = lanes * C + (icblk * 16 * C + j)
                    v = plsc.load_gather(xv, [pos])
                    idxT[j, pl.ds(icblk * 16, 16)] = v

        start_gather(0, 0)

        @pl.loop(0, C, step=2)
        def _(j):
            for p in range(2):
                jj = j + p
                wait_gather(p, jj)

                @pl.when(jj + 1 < C)
                def _():
                    start_gather(1 - p, jj + 1)

                @pl.when(jj >= 2)
                def _():
                    wait_store(p, jj - 2, b)

                transpose_block(p)
                start_store(p, jj, b)

        wait_store(0, C - 2, b)
        wait_store(1, C - 1, b)


def kernel(x, weight):
    xf = x.reshape(-1).astype(jnp.int32)
    y5 = _gather(xf, weight)
    return y5.transpose(2, 4, 0, 1, 3).reshape(R, C, D)
